# Initial kernel scaffold; baseline (speedup 1.0000x reference)
import jax
import jax.numpy as jnp
from jax.experimental import pallas as pl


def kernel(x, edge_index, edge_attr, stemtypes, stem_node_idx, stem_batch, node_batch, blockemb, stememb, bondemb, Wb1, bb1, Wb2, bb2, conv_root, conv_bias, W_ih, b_ih, W_hh, b_hh, Ws1, bs1, Ws2, bs2, Ws3, bs3, Wg1, bg1, Wg2, bg2):
    raise NotImplementedError("write your pallas kernel here")


# speedup vs baseline: 10.5569x; 10.5569x over previous
import functools

import jax
import jax.numpy as jnp
from jax import lax
from jax.experimental import pallas as pl
from jax.experimental.pallas import tpu as pltpu
from jax.experimental.pallas import tpu_sc as plsc

_NC, _NS = 2, 16
_NW = _NC * _NS
_N, _E, _S, _G = 10000, 320000, 20000, 128
_K = 20
_NPAD = 10112
_TLEN = _NPAD * _K
_TSLICE = _TLEN // _NS
_EPW = 10112
_ECH = _EPW // 128
_EPAD = _EPW * _NW
_SPAD = 20480
_SPW = _SPAD // _NW
_NB = _NPAD // 128
_SB = _SPAD // 128
_EB = _EPAD // 128 // _NB

_f32 = jnp.float32
_i32 = jnp.int32


def _lk(v):
    return jnp.where(v >= 0, v, 0.01 * v)


def _prep_body(x_ref, src_ref, dst_ref, a0_ref, a1_ref, blk_ref, wb1t_ref,
               bb1_ref, wb2t_ref, bb2_ref, bet_ref, h0_ref, p0_ref, ip_ref,
               it_ref):
    oh = (lax.broadcasted_iota(_i32, (128, 106), 1) == x_ref[...]).astype(_f32)
    emb = jnp.dot(oh, blk_ref[...], preferred_element_type=_f32)
    o1 = _lk(jnp.dot(emb, wb1t_ref[...], preferred_element_type=_f32)
             + bb1_ref[...])
    out0 = jnp.dot(o1, wb2t_ref[...], preferred_element_type=_f32) + bb2_ref[...]
    h0_ref[...] = out0
    p0_ref[...] = jnp.dot(out0, bet_ref[...], preferred_element_type=_f32)
    ip_ref[...] = src_ref[...] * _K + a0_ref[...]
    it_ref[...] = dst_ref[...] * _K + a1_ref[...]


_prep = pl.pallas_call(
    _prep_body,
    grid=(_NB,),
    in_specs=[
        pl.BlockSpec((128, 1), lambda i: (i, 0)),
        pl.BlockSpec((_EB, 128), lambda i: (i, 0)),
        pl.BlockSpec((_EB, 128), lambda i: (i, 0)),
        pl.BlockSpec((_EB, 128), lambda i: (i, 0)),
        pl.BlockSpec((_EB, 128), lambda i: (i, 0)),
        pl.BlockSpec((106, 16), lambda i: (0, 0)),
        pl.BlockSpec((16, 16), lambda i: (0, 0)),
        pl.BlockSpec((1, 16), lambda i: (0, 0)),
        pl.BlockSpec((16, 16), lambda i: (0, 0)),
        pl.BlockSpec((1, 16), lambda i: (0, 0)),
        pl.BlockSpec((16, _K), lambda i: (0, 0)),
    ],
    out_specs=[
        pl.BlockSpec((128, 16), lambda i: (i, 0)),
        pl.BlockSpec((128, _K), lambda i: (i, 0)),
        pl.BlockSpec((_EB, 128), lambda i: (i, 0)),
        pl.BlockSpec((_EB, 128), lambda i: (i, 0)),
    ],
    out_shape=[
        jax.ShapeDtypeStruct((_NPAD, 16), _f32),
        jax.ShapeDtypeStruct((_NPAD, _K), _f32),
        jax.ShapeDtypeStruct((_NB * _EB, 128), _i32),
        jax.ShapeDtypeStruct((_NB * _EB, 128), _i32),
    ],
)


def _edge_body(p_hbm, ip_hbm, it_hbm, z_hbm, tout_hbm, ip_v, it_v, sbuf,
               zbuf, t_sh, gsem0, gsem1, ssem):
    c = lax.axis_index("c")
    s = lax.axis_index("s")
    w = c * _NS + s
    pltpu.sync_copy(ip_hbm.at[pl.ds(w * _EPW, _EPW)], ip_v)
    pltpu.sync_copy(it_hbm.at[pl.ds(w * _EPW, _EPW)], it_v)
    pltpu.sync_copy(z_hbm, zbuf)
    pltpu.sync_copy(zbuf, t_sh.at[pl.ds(s * _TSLICE, _TSLICE)])
    plsc.subcore_barrier()

    gsems = (gsem0, gsem1)

    def gather(j):
        return pltpu.async_copy(
            p_hbm.at[ip_v.at[pl.ds(j * 128, 128)]],
            sbuf.at[pl.ds(j * 128, 128)],
            gsems[j % 2],
        )

    window = 8
    scats = []
    cp_cur = gather(0)
    for j in range(_ECH):
        cp_next = gather(j + 1) if j + 1 < _ECH else None
        cp_cur.wait()
        scats.append(pltpu.async_copy(
            sbuf.at[pl.ds(j * 128, 128)],
            t_sh.at[it_v.at[pl.ds(j * 128, 128)]],
            ssem,
            add=True,
        ))
        if j >= window:
            scats[j - window].wait()
        cp_cur = cp_next
    for sc in scats[max(0, _ECH - window):]:
        sc.wait()
    plsc.subcore_barrier()
    pltpu.sync_copy(t_sh.at[pl.ds(s * _TSLICE, _TSLICE)], zbuf)
    pltpu.sync_copy(zbuf, tout_hbm.at[pl.ds(c * _TLEN + s * _TSLICE, _TSLICE)])


_edge = pl.kernel(
    _edge_body,
    out_type=jax.ShapeDtypeStruct((_NC * _TLEN,), _f32),
    mesh=plsc.VectorSubcoreMesh(core_axis_name="c", subcore_axis_name="s",
                                num_cores=_NC, num_subcores=_NS),
    scratch_types=[
        pltpu.VMEM((_EPW,), _i32),
        pltpu.VMEM((_EPW,), _i32),
        pltpu.VMEM((_EPW,), _f32),
        pltpu.VMEM((_TSLICE,), _f32),
        pltpu.VMEM_SHARED((_TLEN,), _f32),
        pltpu.SemaphoreType.DMA,
        pltpu.SemaphoreType.DMA,
        pltpu.SemaphoreType.DMA,
    ],
)


def _degk_body(t_ref, invd_ref):
    tsum = t_ref[0] + t_ref[1]
    ones = jnp.ones((_K, 1), _f32)
    deg = jnp.dot(tsum, ones, preferred_element_type=_f32)
    invd_ref[...] = 1.0 / jnp.maximum(deg, 1.0)


_degk = pl.pallas_call(
    _degk_body,
    grid=(_NB,),
    in_specs=[pl.BlockSpec((2, 128, _K), lambda i: (0, i, 0))],
    out_specs=pl.BlockSpec((128, 1), lambda i: (i, 0)),
    out_shape=jax.ShapeDtypeStruct((_NPAD, 1), _f32),
)


def _gru_math(h_ref, t0_ref, t1_ref, invd_ref, be_ref, crt_ref, cb_ref,
              wirt_ref, bir_ref, wizt_ref, biz_ref, wint_ref, bin_ref,
              whrt_ref, bhr_ref, whzt_ref, bhz_ref, whnt_ref, bhn_ref):
    t = (t0_ref[...] + t1_ref[...]) * invd_ref[...]
    agg = jnp.dot(t, be_ref[...], preferred_element_type=_f32)
    h = h_ref[...]
    m = _lk(jnp.dot(h, crt_ref[...], preferred_element_type=_f32) + agg
            + cb_ref[...])
    r = jax.nn.sigmoid(jnp.dot(m, wirt_ref[...], preferred_element_type=_f32)
                       + bir_ref[...]
                       + jnp.dot(h, whrt_ref[...], preferred_element_type=_f32)
                       + bhr_ref[...])
    z = jax.nn.sigmoid(jnp.dot(m, wizt_ref[...], preferred_element_type=_f32)
                       + biz_ref[...]
                       + jnp.dot(h, whzt_ref[...], preferred_element_type=_f32)
                       + bhz_ref[...])
    n = jnp.tanh(jnp.dot(m, wint_ref[...], preferred_element_type=_f32)
                 + bin_ref[...]
                 + r * (jnp.dot(h, whnt_ref[...], preferred_element_type=_f32)
                        + bhn_ref[...]))
    return (1.0 - z) * n + z * h


def _dense_body(h_ref, t0_ref, t1_ref, invd_ref, be_ref, bet_ref, crt_ref,
                cb_ref, wirt_ref, bir_ref, wizt_ref, biz_ref, wint_ref,
                bin_ref, whrt_ref, bhr_ref, whzt_ref, bhz_ref, whnt_ref,
                bhn_ref, hout_ref, pout_ref):
    hn = _gru_math(h_ref, t0_ref, t1_ref, invd_ref, be_ref, crt_ref, cb_ref,
                   wirt_ref, bir_ref, wizt_ref, biz_ref, wint_ref, bin_ref,
                   whrt_ref, bhr_ref, whzt_ref, bhz_ref, whnt_ref, bhn_ref)
    hout_ref[...] = hn
    pout_ref[...] = jnp.dot(hn, bet_ref[...], preferred_element_type=_f32)


_w16 = lambda i: (0, 0)
_dense_in_specs = [
    pl.BlockSpec((128, 16), lambda i: (i, 0)),
    pl.BlockSpec((128, _K), lambda i: (i, 0)),
    pl.BlockSpec((128, _K), lambda i: (i, 0)),
    pl.BlockSpec((128, 1), lambda i: (i, 0)),
    pl.BlockSpec((_K, 16), _w16),
    pl.BlockSpec((16, _K), _w16),
    pl.BlockSpec((16, 16), _w16),
    pl.BlockSpec((1, 16), _w16),
] + [pl.BlockSpec((16, 16), _w16), pl.BlockSpec((1, 16), _w16)] * 6

_dense = pl.pallas_call(
    _dense_body,
    grid=(_NB,),
    in_specs=_dense_in_specs,
    out_specs=[
        pl.BlockSpec((128, 16), lambda i: (i, 0)),
        pl.BlockSpec((128, _K), lambda i: (i, 0)),
    ],
    out_shape=[
        jax.ShapeDtypeStruct((_NPAD, 16), _f32),
        jax.ShapeDtypeStruct((_NPAD, _K), _f32),
    ],
)


def _dense_last_body(h_ref, t0_ref, t1_ref, invd_ref, nb_ref, be_ref,
                     bet_ref, crt_ref, cb_ref, wirt_ref, bir_ref, wizt_ref,
                     biz_ref, wint_ref, bin_ref, whrt_ref, bhr_ref, whzt_ref,
                     bhz_ref, whnt_ref, bhn_ref, wg1t_ref, bg1_ref, wg2t_ref,
                     bg2_ref, hout_ref, stop_ref, pool_acc, cnt_acc):
    i = pl.program_id(0)
    hn = _gru_math(h_ref, t0_ref, t1_ref, invd_ref, be_ref, crt_ref, cb_ref,
                   wirt_ref, bir_ref, wizt_ref, biz_ref, wint_ref, bin_ref,
                   whrt_ref, bhr_ref, whzt_ref, bhz_ref, whnt_ref, bhn_ref)
    hout_ref[...] = hn
    oh = (lax.broadcasted_iota(_i32, (128, _G), 1) == nb_ref[...]).astype(_f32)
    pool_blk = lax.dot_general(oh, hn, (((0,), (0,)), ((), ())),
                               preferred_element_type=_f32)
    ones = jnp.ones((128, 1), _f32)
    cnt_blk = lax.dot_general(oh, ones, (((0,), (0,)), ((), ())),
                              preferred_element_type=_f32)

    @pl.when(i == 0)
    def _():
        pool_acc[...] = jnp.zeros_like(pool_acc)
        cnt_acc[...] = jnp.zeros_like(cnt_acc)

    pool_acc[...] += pool_blk
    cnt_acc[...] += cnt_blk

    @pl.when(i == _NB - 1)
    def _():
        pooled = pool_acc[...] / jnp.maximum(cnt_acc[...], 1.0)
        g1 = _lk(jnp.dot(pooled, wg1t_ref[...], preferred_element_type=_f32)
                 + bg1_ref[...])
        stop_ref[...] = (jnp.dot(g1, wg2t_ref[...], preferred_element_type=_f32)
                        + bg2_ref[...])


_dense_last = pl.pallas_call(
    _dense_last_body,
    grid=(_NB,),
    in_specs=_dense_in_specs[:4]
    + [pl.BlockSpec((128, 1), lambda i: (i, 0))]
    + _dense_in_specs[4:]
    + [pl.BlockSpec((16, 16), _w16), pl.BlockSpec((1, 16), _w16),
       pl.BlockSpec((16, 1), _w16), pl.BlockSpec((1, 1), _w16)],
    out_specs=[
        pl.BlockSpec((128, 16), lambda i: (i, 0)),
        pl.BlockSpec((_G, 1), lambda i: (0, 0)),
    ],
    out_shape=[
        jax.ShapeDtypeStruct((_NPAD, 16), _f32),
        jax.ShapeDtypeStruct((_G, 1), _f32),
    ],
    scratch_shapes=[
        pltpu.VMEM((_G, 16), _f32),
        pltpu.VMEM((_G, 1), _f32),
    ],
)


def _sidx_body(st_ref, out_ref):
    out_ref[...] = st_ref[...] * 16 + lax.broadcasted_iota(_i32, (128, 16), 1)


_sidx16 = pl.pallas_call(
    _sidx_body,
    grid=(_SB,),
    in_specs=[pl.BlockSpec((128, 1), lambda i: (i, 0))],
    out_specs=pl.BlockSpec((128, 16), lambda i: (i, 0)),
    out_shape=jax.ShapeDtypeStruct((_SPAD, 16), _i32),
)


_SGW = _SPAD * 16 // _NW


def _sgath_body(h_hbm, sidx_hbm, gout_hbm, sidx_v, sbuf, gsem0, gsem1):
    c = lax.axis_index("c")
    s = lax.axis_index("s")
    w = c * _NS + s
    pltpu.sync_copy(sidx_hbm.at[pl.ds(w * _SGW, _SGW)], sidx_v)
    gsems = (gsem0, gsem1)

    def gather(j):
        return pltpu.async_copy(
            h_hbm.at[sidx_v.at[pl.ds(j * 128, 128)]],
            sbuf.at[pl.ds(j * 128, 128)],
            gsems[j % 2],
        )

    cp_cur = gather(0)
    for j in range(_SGW // 128):
        cp_next = gather(j + 1) if j + 1 < _SGW // 128 else None
        cp_cur.wait()
        cp_cur = cp_next
    pltpu.sync_copy(sbuf, gout_hbm.at[pl.ds(w * _SGW, _SGW)])


_sgath = pl.kernel(
    _sgath_body,
    out_type=jax.ShapeDtypeStruct((_SPAD * 16,), _f32),
    mesh=plsc.VectorSubcoreMesh(core_axis_name="c", subcore_axis_name="s",
                                num_cores=_NC, num_subcores=_NS),
    scratch_types=[
        pltpu.VMEM((_SGW,), _i32),
        pltpu.VMEM((_SGW,), _f32),
        pltpu.SemaphoreType.DMA,
        pltpu.SemaphoreType.DMA,
    ],
)


def _stem_body(g_ref, st_ref, semb_ref, w1at_ref, w1bt_ref, b1_ref, w2t_ref,
               b2_ref, w3t_ref, b3_ref, out_ref):
    oh = (lax.broadcasted_iota(_i32, (128, 21), 1) == st_ref[...]).astype(_f32)
    semb = jnp.dot(oh, semb_ref[...], preferred_element_type=_f32)
    s1 = _lk(jnp.dot(g_ref[...], w1at_ref[...], preferred_element_type=_f32)
             + jnp.dot(semb, w1bt_ref[...], preferred_element_type=_f32)
             + b1_ref[...])
    s2 = _lk(jnp.dot(s1, w2t_ref[...], preferred_element_type=_f32)
             + b2_ref[...])
    out_ref[...] = (jnp.dot(s2, w3t_ref[...], preferred_element_type=_f32)
                    + b3_ref[...])


_stem = pl.pallas_call(
    _stem_body,
    grid=(_SB,),
    in_specs=[
        pl.BlockSpec((128, 16), lambda i: (i, 0)),
        pl.BlockSpec((128, 1), lambda i: (i, 0)),
        pl.BlockSpec((21, 16), _w16),
        pl.BlockSpec((16, 16), _w16),
        pl.BlockSpec((16, 16), _w16),
        pl.BlockSpec((1, 16), _w16),
        pl.BlockSpec((16, 16), _w16),
        pl.BlockSpec((1, 16), _w16),
        pl.BlockSpec((16, 105), _w16),
        pl.BlockSpec((1, 105), _w16),
    ],
    out_specs=pl.BlockSpec((128, 105), lambda i: (i, 0)),
    out_shape=jax.ShapeDtypeStruct((_SPAD, 105), _f32),
)


def kernel(x, edge_index, edge_attr, stemtypes, stem_node_idx, stem_batch,
           node_batch, blockemb, stememb, bondemb, Wb1, bb1, Wb2, bb2,
           conv_root, conv_bias, W_ih, b_ih, W_hh, b_hh, Ws1, bs1, Ws2, bs2,
           Ws3, bs3, Wg1, bg1, Wg2, bg2):
    x_p = jnp.pad(x.astype(_i32), (0, _NPAD - _N)).reshape(_NPAD, 1)
    src = edge_index[0].astype(_i32)
    dst = edge_index[1].astype(_i32)
    a0 = edge_attr[:, 0].astype(_i32)
    a1 = edge_attr[:, 1].astype(_i32)
    epad = _EPAD - _E
    erows = _NB * _EB
    src_p = jnp.pad(src, (0, epad), constant_values=_NPAD - 1).reshape(erows, 128)
    dst_p = jnp.pad(dst, (0, epad), constant_values=_NPAD - 1).reshape(erows, 128)
    a0_p = jnp.pad(a0, (0, epad)).reshape(erows, 128)
    a1_p = jnp.pad(a1, (0, epad)).reshape(erows, 128)

    r2 = lambda b: b.astype(_f32).reshape(1, -1)
    h0, P0, ipf, itf = _prep(
        x_p, src_p, dst_p, a0_p, a1_p, blockemb, Wb1.T, r2(bb1), Wb2.T,
        r2(bb2), bondemb.T)
    ipw = ipf.reshape(_EPAD)
    itw = itf.reshape(_EPAD)

    zslice = jnp.zeros((_TSLICE,), _f32)
    ones_p = jnp.ones((_TLEN,), _f32)
    t_cnt = _edge(ones_p, ipw, itw, zslice)
    invd = _degk(t_cnt.reshape(_NC, _NPAD, _K))

    wih = [W_ih[i * 16:(i + 1) * 16].T for i in range(3)]
    bih = [r2(b_ih[i * 16:(i + 1) * 16]) for i in range(3)]
    whh = [W_hh[i * 16:(i + 1) * 16].T for i in range(3)]
    bhh = [r2(b_hh[i * 16:(i + 1) * 16]) for i in range(3)]
    dense_w = (bondemb, bondemb.T, conv_root.T, r2(conv_bias),
               wih[0], bih[0], wih[1], bih[1], wih[2], bih[2],
               whh[0], bhh[0], whh[1], bhh[1], whh[2], bhh[2])

    h, P = h0, P0
    for _ in range(5):
        t = _edge(P.reshape(_TLEN), ipw, itw, zslice)
        t = t.reshape(_NC, _NPAD, _K)
        h, P = _dense(h, t[0], t[1], invd, *dense_w)

    t = _edge(P.reshape(_TLEN), ipw, itw, zslice)
    t = t.reshape(_NC, _NPAD, _K)
    nb_p = jnp.pad(node_batch.astype(_i32), (0, _NPAD - _N),
                   constant_values=_G).reshape(_NPAD, 1)
    h, stop_pred = _dense_last(h, t[0], t[1], invd, nb_p, *dense_w,
                               Wg1.T, r2(bg1), Wg2.T, r2(bg2))

    sidx = jnp.pad(stem_node_idx.astype(_i32),
                   (0, _SPAD - _S)).reshape(_SPAD, 1)
    sidx16 = _sidx16(sidx).reshape(_SPAD * 16)
    gath = _sgath(h.reshape(_NPAD * 16), sidx16).reshape(_SPAD, 16)
    st_p = jnp.pad(stemtypes.astype(_i32), (0, _SPAD - _S)).reshape(_SPAD, 1)
    stem_pad = _stem(gath, st_p, stememb, Ws1[:, :16].T, Ws1[:, 16:].T,
                     r2(bs1), Ws2.T, r2(bs2), Ws3.T, r2(bs3))
    return stem_pad[:_S], stop_pred, stem_batch


# speedup vs baseline: 20.6692x; 1.9579x over previous
import functools

import jax
import jax.numpy as jnp
from jax import lax
from jax.experimental import pallas as pl
from jax.experimental.pallas import tpu as pltpu
from jax.experimental.pallas import tpu_sc as plsc

_NC, _NS = 2, 16
_NW = _NC * _NS
_N, _E, _S, _G = 10000, 320000, 20000, 128
_K = 20
_NPAD = 10112
_TLEN = _NPAD * _K
_TSLICE = _TLEN // _NS
_EPW = 10112
_ECH = _EPW // 128
_EPAD = _EPW * _NW
_SPAD = 20480
_SPW = _SPAD // _NW
_SGW = _SPW * 16

_f32 = jnp.float32
_i32 = jnp.int32


def _lk(v):
    return jnp.where(v >= 0, v, 0.01 * v)


def _prep_body(x_ref, src_ref, dst_ref, a0_ref, a1_ref, blk_ref, wb1t_ref,
               bb1_ref, wb2t_ref, bb2_ref, bet_ref, h0_ref, p0_ref, ip_ref,
               it_ref):
    oh = (lax.broadcasted_iota(_i32, (_NPAD, 106), 1) == x_ref[...]).astype(_f32)
    emb = jnp.dot(oh, blk_ref[...], preferred_element_type=_f32)
    o1 = _lk(jnp.dot(emb, wb1t_ref[...], preferred_element_type=_f32)
             + bb1_ref[...])
    out0 = jnp.dot(o1, wb2t_ref[...], preferred_element_type=_f32) + bb2_ref[...]
    h0_ref[...] = out0
    p0_ref[...] = jnp.dot(out0, bet_ref[...], preferred_element_type=_f32)
    ip_ref[...] = src_ref[...] * _K + a0_ref[...]
    it_ref[...] = dst_ref[...] * _K + a1_ref[...]


_ERows = _EPAD // 128

_prep = pl.pallas_call(
    _prep_body,
    out_shape=[
        jax.ShapeDtypeStruct((_NPAD, 16), _f32),
        jax.ShapeDtypeStruct((_NPAD, _K), _f32),
        jax.ShapeDtypeStruct((_ERows, 128), _i32),
        jax.ShapeDtypeStruct((_ERows, 128), _i32),
    ],
)


def _edge_body(p_hbm, ip_hbm, it_hbm, z_hbm, tout_hbm, ip_v, it_v, sbuf,
               zbuf, t_sh, gsem0, gsem1, gsem2, gsem3, ssem):
    c = lax.axis_index("c")
    s = lax.axis_index("s")
    w = c * _NS + s
    pltpu.sync_copy(ip_hbm.at[pl.ds(w * _EPW, _EPW)], ip_v)
    pltpu.sync_copy(it_hbm.at[pl.ds(w * _EPW, _EPW)], it_v)
    pltpu.sync_copy(z_hbm, zbuf)
    pltpu.sync_copy(zbuf, t_sh.at[pl.ds(s * _TSLICE, _TSLICE)])
    plsc.subcore_barrier()

    gsems = (gsem0, gsem1, gsem2, gsem3)
    depth = 4

    def gather(j):
        return pltpu.async_copy(
            p_hbm.at[ip_v.at[pl.ds(j * 128, 128)]],
            sbuf.at[pl.ds(j * 128, 128)],
            gsems[j % depth],
        )

    window = 8
    pending = [gather(j) for j in range(depth - 1)]
    scats = []
    for j in range(_ECH):
        if j + depth - 1 < _ECH:
            pending.append(gather(j + depth - 1))
        pending.pop(0).wait()
        scats.append(pltpu.async_copy(
            sbuf.at[pl.ds(j * 128, 128)],
            t_sh.at[it_v.at[pl.ds(j * 128, 128)]],
            ssem,
            add=True,
        ))
        if j >= window:
            scats[j - window].wait()
    for sc in scats[max(0, _ECH - window):]:
        sc.wait()
    plsc.subcore_barrier()
    pltpu.sync_copy(t_sh.at[pl.ds(s * _TSLICE, _TSLICE)], zbuf)
    pltpu.sync_copy(zbuf, tout_hbm.at[pl.ds(c * _TLEN + s * _TSLICE, _TSLICE)])


_edge = pl.kernel(
    _edge_body,
    out_type=jax.ShapeDtypeStruct((_NC * _TLEN,), _f32),
    mesh=plsc.VectorSubcoreMesh(core_axis_name="c", subcore_axis_name="s",
                                num_cores=_NC, num_subcores=_NS),
    scratch_types=[
        pltpu.VMEM((_EPW,), _i32),
        pltpu.VMEM((_EPW,), _i32),
        pltpu.VMEM((_EPW,), _f32),
        pltpu.VMEM((_TSLICE,), _f32),
        pltpu.VMEM_SHARED((_TLEN,), _f32),
        pltpu.SemaphoreType.DMA,
        pltpu.SemaphoreType.DMA,
        pltpu.SemaphoreType.DMA,
        pltpu.SemaphoreType.DMA,
        pltpu.SemaphoreType.DMA,
    ],
)


def _degk_body(t_ref, invd_ref):
    tsum = t_ref[0] + t_ref[1]
    ones = jnp.ones((_K, 1), _f32)
    deg = jnp.dot(tsum, ones, preferred_element_type=_f32)
    invd_ref[...] = 1.0 / jnp.maximum(deg, 1.0)


_degk = pl.pallas_call(
    _degk_body,
    out_shape=jax.ShapeDtypeStruct((_NPAD, 1), _f32),
)


def _gru_math(h_ref, t_ref, invd_ref, be_ref, crt_ref, cb_ref,
              wirt_ref, bir_ref, wizt_ref, biz_ref, wint_ref, bin_ref,
              whrt_ref, bhr_ref, whzt_ref, bhz_ref, whnt_ref, bhn_ref):
    t = (t_ref[0] + t_ref[1]) * invd_ref[...]
    agg = jnp.dot(t, be_ref[...], preferred_element_type=_f32)
    h = h_ref[...]
    m = _lk(jnp.dot(h, crt_ref[...], preferred_element_type=_f32) + agg
            + cb_ref[...])
    r = jax.nn.sigmoid(jnp.dot(m, wirt_ref[...], preferred_element_type=_f32)
                       + bir_ref[...]
                       + jnp.dot(h, whrt_ref[...], preferred_element_type=_f32)
                       + bhr_ref[...])
    z = jax.nn.sigmoid(jnp.dot(m, wizt_ref[...], preferred_element_type=_f32)
                       + biz_ref[...]
                       + jnp.dot(h, whzt_ref[...], preferred_element_type=_f32)
                       + bhz_ref[...])
    n = jnp.tanh(jnp.dot(m, wint_ref[...], preferred_element_type=_f32)
                 + bin_ref[...]
                 + r * (jnp.dot(h, whnt_ref[...], preferred_element_type=_f32)
                        + bhn_ref[...]))
    return (1.0 - z) * n + z * h


def _dense_body(h_ref, t_ref, invd_ref, be_ref, bet_ref, crt_ref,
                cb_ref, wirt_ref, bir_ref, wizt_ref, biz_ref, wint_ref,
                bin_ref, whrt_ref, bhr_ref, whzt_ref, bhz_ref, whnt_ref,
                bhn_ref, hout_ref, pout_ref):
    hn = _gru_math(h_ref, t_ref, invd_ref, be_ref, crt_ref, cb_ref,
                   wirt_ref, bir_ref, wizt_ref, biz_ref, wint_ref, bin_ref,
                   whrt_ref, bhr_ref, whzt_ref, bhz_ref, whnt_ref, bhn_ref)
    hout_ref[...] = hn
    pout_ref[...] = jnp.dot(hn, bet_ref[...], preferred_element_type=_f32)


_dense = pl.pallas_call(
    _dense_body,
    out_shape=[
        jax.ShapeDtypeStruct((_NPAD, 16), _f32),
        jax.ShapeDtypeStruct((_NPAD, _K), _f32),
    ],
)


def _dense_last_body(h_ref, t_ref, invd_ref, nb_ref, be_ref,
                     bet_ref, crt_ref, cb_ref, wirt_ref, bir_ref, wizt_ref,
                     biz_ref, wint_ref, bin_ref, whrt_ref, bhr_ref, whzt_ref,
                     bhz_ref, whnt_ref, bhn_ref, wg1t_ref, bg1_ref, wg2t_ref,
                     bg2_ref, hout_ref, stop_ref):
    hn = _gru_math(h_ref, t_ref, invd_ref, be_ref, crt_ref, cb_ref,
                   wirt_ref, bir_ref, wizt_ref, biz_ref, wint_ref, bin_ref,
                   whrt_ref, bhr_ref, whzt_ref, bhz_ref, whnt_ref, bhn_ref)
    hout_ref[...] = hn
    oh = (lax.broadcasted_iota(_i32, (_NPAD, _G), 1) == nb_ref[...]).astype(_f32)
    pooled = lax.dot_general(oh, hn, (((0,), (0,)), ((), ())),
                             preferred_element_type=_f32)
    ones = jnp.ones((_NPAD, 1), _f32)
    cnt = lax.dot_general(oh, ones, (((0,), (0,)), ((), ())),
                          preferred_element_type=_f32)
    pooled = pooled / jnp.maximum(cnt, 1.0)
    g1 = _lk(jnp.dot(pooled, wg1t_ref[...], preferred_element_type=_f32)
             + bg1_ref[...])
    stop_ref[...] = (jnp.dot(g1, wg2t_ref[...], preferred_element_type=_f32)
                     + bg2_ref[...])


_dense_last = pl.pallas_call(
    _dense_last_body,
    out_shape=[
        jax.ShapeDtypeStruct((_NPAD, 16), _f32),
        jax.ShapeDtypeStruct((_G, 1), _f32),
    ],
)


def _sgath_body(h_hbm, sidx_hbm, gout_hbm, sidx_v, idx16_v, sbuf, gsem0,
                gsem1, gsem2, gsem3):
    c = lax.axis_index("c")
    s = lax.axis_index("s")
    w = c * _NS + s
    pltpu.sync_copy(sidx_hbm.at[pl.ds(w * _SPW, _SPW)], sidx_v)
    lanes = lax.iota(_i32, 16)
    for g in range(_SPW // 16):
        snv = sidx_v[pl.ds(g * 16, 16)]
        for l in range(16):
            idx16_v[pl.ds((g * 16 + l) * 16, 16)] = snv[l] * 16 + lanes

    gsems = (gsem0, gsem1, gsem2, gsem3)
    depth = 4
    nch = _SGW // 128

    def gather(j):
        return pltpu.async_copy(
            h_hbm.at[idx16_v.at[pl.ds(j * 128, 128)]],
            sbuf.at[pl.ds(j * 128, 128)],
            gsems[j % depth],
        )

    pending = [gather(j) for j in range(depth - 1)]
    for j in range(nch):
        if j + depth - 1 < nch:
            pending.append(gather(j + depth - 1))
        pending.pop(0).wait()
    pltpu.sync_copy(sbuf, gout_hbm.at[pl.ds(w * _SGW, _SGW)])


_sgath = pl.kernel(
    _sgath_body,
    out_type=jax.ShapeDtypeStruct((_SPAD * 16,), _f32),
    mesh=plsc.VectorSubcoreMesh(core_axis_name="c", subcore_axis_name="s",
                                num_cores=_NC, num_subcores=_NS),
    scratch_types=[
        pltpu.VMEM((_SPW,), _i32),
        pltpu.VMEM((_SGW,), _i32),
        pltpu.VMEM((_SGW,), _f32),
        pltpu.SemaphoreType.DMA,
        pltpu.SemaphoreType.DMA,
        pltpu.SemaphoreType.DMA,
        pltpu.SemaphoreType.DMA,
    ],
)


def _stem_body(g_ref, st_ref, semb_ref, w1at_ref, w1bt_ref, b1_ref, w2t_ref,
               b2_ref, w3t_ref, b3_ref, out_ref):
    oh = (lax.broadcasted_iota(_i32, (_SPAD, 21), 1) == st_ref[...]).astype(_f32)
    semb = jnp.dot(oh, semb_ref[...], preferred_element_type=_f32)
    s1 = _lk(jnp.dot(g_ref[...], w1at_ref[...], preferred_element_type=_f32)
             + jnp.dot(semb, w1bt_ref[...], preferred_element_type=_f32)
             + b1_ref[...])
    s2 = _lk(jnp.dot(s1, w2t_ref[...], preferred_element_type=_f32)
             + b2_ref[...])
    out_ref[...] = (jnp.dot(s2, w3t_ref[...], preferred_element_type=_f32)
                    + b3_ref[...])


_stem = pl.pallas_call(
    _stem_body,
    out_shape=jax.ShapeDtypeStruct((_SPAD, 105), _f32),
)


def kernel(x, edge_index, edge_attr, stemtypes, stem_node_idx, stem_batch,
           node_batch, blockemb, stememb, bondemb, Wb1, bb1, Wb2, bb2,
           conv_root, conv_bias, W_ih, b_ih, W_hh, b_hh, Ws1, bs1, Ws2, bs2,
           Ws3, bs3, Wg1, bg1, Wg2, bg2):
    x_p = jnp.pad(x.astype(_i32), (0, _NPAD - _N)).reshape(_NPAD, 1)
    src = edge_index[0].astype(_i32)
    dst = edge_index[1].astype(_i32)
    a0 = edge_attr[:, 0].astype(_i32)
    a1 = edge_attr[:, 1].astype(_i32)
    epad = _EPAD - _E
    src_p = jnp.pad(src, (0, epad), constant_values=_NPAD - 1).reshape(_ERows, 128)
    dst_p = jnp.pad(dst, (0, epad), constant_values=_NPAD - 1).reshape(_ERows, 128)
    a0_p = jnp.pad(a0, (0, epad)).reshape(_ERows, 128)
    a1_p = jnp.pad(a1, (0, epad)).reshape(_ERows, 128)

    r2 = lambda b: b.astype(_f32).reshape(1, -1)
    h0, P0, ipf, itf = _prep(
        x_p, src_p, dst_p, a0_p, a1_p, blockemb, Wb1.T, r2(bb1), Wb2.T,
        r2(bb2), bondemb.T)
    ipw = ipf.reshape(_EPAD)
    itw = itf.reshape(_EPAD)

    zslice = jnp.zeros((_TSLICE,), _f32)
    ones_p = jnp.ones((_TLEN,), _f32)
    t_cnt = _edge(ones_p, ipw, itw, zslice).reshape(_NC, _NPAD, _K)
    invd = _degk(t_cnt)

    wih = [W_ih[i * 16:(i + 1) * 16].T for i in range(3)]
    bih = [r2(b_ih[i * 16:(i + 1) * 16]) for i in range(3)]
    whh = [W_hh[i * 16:(i + 1) * 16].T for i in range(3)]
    bhh = [r2(b_hh[i * 16:(i + 1) * 16]) for i in range(3)]
    dense_w = (bondemb, bondemb.T, conv_root.T, r2(conv_bias),
               wih[0], bih[0], wih[1], bih[1], wih[2], bih[2],
               whh[0], bhh[0], whh[1], bhh[1], whh[2], bhh[2])

    h, P = h0, P0
    for _ in range(5):
        t = _edge(P.reshape(_TLEN), ipw, itw, zslice).reshape(_NC, _NPAD, _K)
        h, P = _dense(h, t, invd, *dense_w)

    t = _edge(P.reshape(_TLEN), ipw, itw, zslice).reshape(_NC, _NPAD, _K)
    nb_p = jnp.pad(node_batch.astype(_i32), (0, _NPAD - _N),
                   constant_values=_G).reshape(_NPAD, 1)
    h, stop_pred = _dense_last(h, t, invd, nb_p, *dense_w,
                               Wg1.T, r2(bg1), Wg2.T, r2(bg2))

    sidx = jnp.pad(stem_node_idx.astype(_i32), (0, _SPAD - _S))
    gath = _sgath(h.reshape(_NPAD * 16), sidx).reshape(_SPAD, 16)
    st_p = jnp.pad(stemtypes.astype(_i32), (0, _SPAD - _S)).reshape(_SPAD, 1)
    stem_pad = _stem(gath, st_p, stememb, Ws1[:, :16].T, Ws1[:, 16:].T,
                     r2(bs1), Ws2.T, r2(bs2), Ws3.T, r2(bs3))
    return stem_pad[:_S], stop_pred, stem_batch
